# baseline (speedup 1.0000x reference)
import jax
import jax.numpy as jnp
from jax.experimental import pallas as pl

_NUM_EMBEDDINGS = 1024
_EMBEDDING_DIM = 32
_COMMITMENT_COST = 0.25


def _rowsum_sq_ref_order(x):
    x2 = x * x
    a = x2[:, 0:8] + x2[:, 8:16]
    a = a + x2[:, 16:24]
    a = a + x2[:, 24:32]
    a = a[:, 0:4] + a[:, 4:8]
    a = a[:, 0:2] + a[:, 2:4]
    return a[:, 0:1] + a[:, 1:2]


def _vq_kernel(lat_ref, emb_ref, e2_ref, out_ref, loss_ref):
    b = pl.program_id(0)
    lat = lat_ref[0]
    flat = lat.T
    emb = emb_ref[...]
    e2 = e2_ref[...]
    f2 = _rowsum_sq_ref_order(flat)
    d = (f2 + e2) - 2.0 * jnp.dot(flat, emb.T,
                                  preferred_element_type=jnp.float32)
    iota = jax.lax.broadcasted_iota(jnp.int32, d.shape, 1)
    dmin = jnp.min(d, axis=1, keepdims=True)
    idx = jnp.min(jnp.where(d == dmin, iota, d.shape[1]), axis=1)
    onehot = (iota == idx[:, None]).astype(jnp.float32)
    q = jnp.dot(onehot, emb, preferred_element_type=jnp.float32)
    diff = flat - q
    sq = jnp.sum(diff * diff).reshape(1, 1)

    @pl.when(b == 0)
    def _init():
        loss_ref[...] = jnp.zeros((1, 1), jnp.float32)

    loss_ref[...] += sq
    out_ref[0] = q.T


def kernel(latents, embedding):
    B, C, H, W = latents.shape
    K = embedding.shape[0]
    HW = H * W
    lat3 = latents.reshape(B, C, HW)
    e2_all = jnp.sum(embedding ** 2, axis=1).reshape(1, K)

    out, loss_sum = pl.pallas_call(
        _vq_kernel,
        grid=(B,),
        in_specs=[
            pl.BlockSpec((1, C, HW), lambda b: (b, 0, 0)),
            pl.BlockSpec((K, C), lambda b: (0, 0)),
            pl.BlockSpec((1, K), lambda b: (0, 0)),
        ],
        out_specs=[
            pl.BlockSpec((1, C, HW), lambda b: (b, 0, 0)),
            pl.BlockSpec((1, 1), lambda b: (0, 0)),
        ],
        out_shape=[
            jax.ShapeDtypeStruct((B, C, HW), jnp.float32),
            jax.ShapeDtypeStruct((1, 1), jnp.float32),
        ],
    )(lat3, embedding, e2_all)
    n_elems = B * C * HW
    vq_loss = (1.0 + _COMMITMENT_COST) * loss_sum[0, 0] / n_elems
    return out.reshape(B, C, H, W), vq_loss


# speedup vs baseline: 2.4168x; 2.4168x over previous
import jax
import jax.numpy as jnp
from jax.experimental import pallas as pl

_NUM_EMBEDDINGS = 1024
_EMBEDDING_DIM = 32
_COMMITMENT_COST = 0.25


def _vq_kernel(lat_ref, emb_ref, e2_ref, out_ref, loss_ref):
    b = pl.program_id(0)
    lat = lat_ref[0]
    emb = emb_ref[...]
    e2 = e2_ref[...]
    lat2 = lat * lat
    a = lat2[0:8] + lat2[8:16]
    a = a + lat2[16:24]
    a = a + lat2[24:32]
    a = a[0:4] + a[4:8]
    a = a[0:2] + a[2:4]
    f2 = a[0:1] + a[1:2]
    dt = (f2 + e2) - 2.0 * jnp.dot(emb, lat,
                                   preferred_element_type=jnp.float32)
    iota = jax.lax.broadcasted_iota(jnp.int32, dt.shape, 0)
    dmin = jnp.min(dt, axis=0, keepdims=True)
    idx = jnp.min(jnp.where(dt == dmin, iota, dt.shape[0]),
                  axis=0, keepdims=True)
    onehot = (iota == idx).astype(jnp.float32)
    qt = jnp.dot(emb.T, onehot,
                 preferred_element_type=jnp.float32)
    diff = lat - qt
    sq = jnp.sum(diff * diff).reshape(1, 1)

    @pl.when(b == 0)
    def _init():
        loss_ref[...] = jnp.zeros((1, 1), jnp.float32)

    loss_ref[...] += sq
    out_ref[0] = qt


def kernel(latents, embedding):
    B, C, H, W = latents.shape
    K = embedding.shape[0]
    HW = H * W
    lat3 = latents.reshape(B, C, HW)
    e2_all = jnp.sum(embedding ** 2, axis=1).reshape(K, 1)

    out, loss_sum = pl.pallas_call(
        _vq_kernel,
        grid=(B,),
        in_specs=[
            pl.BlockSpec((1, C, HW), lambda b: (b, 0, 0)),
            pl.BlockSpec((K, C), lambda b: (0, 0)),
            pl.BlockSpec((K, 1), lambda b: (0, 0)),
        ],
        out_specs=[
            pl.BlockSpec((1, C, HW), lambda b: (b, 0, 0)),
            pl.BlockSpec((1, 1), lambda b: (0, 0)),
        ],
        out_shape=[
            jax.ShapeDtypeStruct((B, C, HW), jnp.float32),
            jax.ShapeDtypeStruct((1, 1), jnp.float32),
        ],
    )(lat3, embedding, e2_all)
    n_elems = B * C * HW
    vq_loss = (1.0 + _COMMITMENT_COST) * loss_sum[0, 0] / n_elems
    return out.reshape(B, C, H, W), vq_loss


# speedup vs baseline: 2.4959x; 1.0327x over previous
import jax
import jax.numpy as jnp
from jax.experimental import pallas as pl

_NUM_EMBEDDINGS = 1024
_EMBEDDING_DIM = 32
_COMMITMENT_COST = 0.25


_BATCHES_PER_STEP = 2


def _vq_kernel(lat_ref, emb_ref, e2_ref, out_ref, loss_ref):
    b = pl.program_id(0)
    emb = emb_ref[...]
    e2 = e2_ref[...]
    sq = jnp.zeros((1, 1), jnp.float32)
    for i in range(_BATCHES_PER_STEP):
        lat = lat_ref[i]
        lat2 = lat * lat
        a = lat2[0:8] + lat2[8:16]
        a = a + lat2[16:24]
        a = a + lat2[24:32]
        a = a[0:4] + a[4:8]
        a = a[0:2] + a[2:4]
        f2 = a[0:1] + a[1:2]
        dt = (f2 + e2) - 2.0 * jnp.dot(emb, lat,
                                       preferred_element_type=jnp.float32)
        iota = jax.lax.broadcasted_iota(jnp.int32, dt.shape, 0)
        dmin = jnp.min(dt, axis=0, keepdims=True)
        idx = jnp.min(jnp.where(dt == dmin, iota, dt.shape[0]),
                      axis=0, keepdims=True)
        onehot = (iota == idx).astype(jnp.float32)
        qt = jnp.dot(emb.T, onehot,
                     preferred_element_type=jnp.float32)
        diff = lat - qt
        sq = sq + jnp.sum(diff * diff).reshape(1, 1)
        out_ref[i] = qt

    @pl.when(b == 0)
    def _init():
        loss_ref[...] = jnp.zeros((1, 1), jnp.float32)

    loss_ref[...] += sq


def kernel(latents, embedding):
    B, C, H, W = latents.shape
    K = embedding.shape[0]
    HW = H * W
    lat3 = latents.reshape(B, C, HW)
    e2_all = jnp.sum(embedding ** 2, axis=1).reshape(K, 1)

    nb = _BATCHES_PER_STEP
    out, loss_sum = pl.pallas_call(
        _vq_kernel,
        grid=(B // nb,),
        in_specs=[
            pl.BlockSpec((nb, C, HW), lambda b: (b, 0, 0)),
            pl.BlockSpec((K, C), lambda b: (0, 0)),
            pl.BlockSpec((K, 1), lambda b: (0, 0)),
        ],
        out_specs=[
            pl.BlockSpec((nb, C, HW), lambda b: (b, 0, 0)),
            pl.BlockSpec((1, 1), lambda b: (0, 0)),
        ],
        out_shape=[
            jax.ShapeDtypeStruct((B, C, HW), jnp.float32),
            jax.ShapeDtypeStruct((1, 1), jnp.float32),
        ],
    )(lat3, embedding, e2_all)
    n_elems = B * C * HW
    vq_loss = (1.0 + _COMMITMENT_COST) * loss_sum[0, 0] / n_elems
    return out.reshape(B, C, H, W), vq_loss
